# baseline (speedup 1.0000x reference)
import functools

import jax
import jax.numpy as jnp
from jax import lax
from jax.experimental import pallas as pl
from jax.experimental.pallas import tpu as pltpu
from jax.experimental.pallas import tpu_sc as plsc

_E = 256
_N = 8192
_R = 8192
_TI = 1024
_CJ = 8192
_NC = 2
_NS = 16
_NW = _NC * _NS
_BPW = _R // _NW
_ICH = 128
_KCH = _BPW // _ICH


def _argmin_kernel(f_ref, e_ref, sf2_ref, se2_ref, idx_ref):
    mm = jnp.dot(f_ref[...], e_ref[...], preferred_element_type=jnp.float32)
    d = (sf2_ref[...] + se2_ref[...]) - mm
    m = jnp.min(d, axis=1, keepdims=True)
    cols = lax.broadcasted_iota(jnp.int32, (_TI, _N), 1)
    idx_ref[...] = jnp.min(jnp.where(d == m, cols, _N), axis=1, keepdims=True)


def _argmin_call(flat, embeddings, sf, se, interpret=False):
    return pl.pallas_call(
        _argmin_kernel,
        grid=(_R // _TI,),
        in_specs=[
            pl.BlockSpec((_TI, _E), lambda i: (i, 0)),
            pl.BlockSpec((_E, _N), lambda i: (0, 0)),
            pl.BlockSpec((_TI, 1), lambda i: (i, 0)),
            pl.BlockSpec((1, _N), lambda i: (0, 0)),
        ],
        out_specs=pl.BlockSpec((_TI, 1), lambda i: (i, 0)),
        out_shape=jax.ShapeDtypeStruct((_R, 1), jnp.int32),
        compiler_params=pltpu.CompilerParams(
            dimension_semantics=("arbitrary",),
        ),
        interpret=interpret,
    )(flat, embeddings, sf, se)


def _sc_gather(table, idx3):
    mesh = plsc.VectorSubcoreMesh(core_axis_name="c", subcore_axis_name="s",
                                  num_cores=_NC, num_subcores=_NS)

    @functools.partial(
        pl.kernel,
        out_type=jax.ShapeDtypeStruct((_R, _E), jnp.float32),
        mesh=mesh,
        scratch_types=[
            pltpu.VMEM((_KCH, _ICH), jnp.int32),
            pltpu.VMEM((_BPW, _E), jnp.float32),
            pltpu.SemaphoreType.DMA,
        ],
    )
    def gather_kernel(table_hbm, idx_hbm, out_hbm, idx_v, rows_v, sem):
        wid = lax.axis_index("s") * _NC + lax.axis_index("c")
        pltpu.sync_copy(idx_hbm.at[wid], idx_v)
        copies = [
            pltpu.async_copy(table_hbm.at[idx_v.at[k]],
                             rows_v.at[pl.ds(k * _ICH, _ICH)], sem)
            for k in range(_KCH)
        ]
        for cp in copies:
            cp.wait()
        pltpu.sync_copy(rows_v, out_hbm.at[pl.ds(wid * _BPW, _BPW)])

    return gather_kernel(table, idx3)


def kernel(x, embeddings):
    flat = x.reshape(-1, _E)
    sf2 = jnp.sum(flat ** 2, axis=1, keepdims=True) * 0.5
    se2 = jnp.sum(embeddings ** 2, axis=0, keepdims=True) * 0.5
    idx = _argmin_call(flat, embeddings, sf2, se2)
    q = _sc_gather(embeddings.T, idx.reshape(_NW, _KCH, _ICH))
    return q.reshape(x.shape)


# speedup vs baseline: 1.0115x; 1.0115x over previous
import functools

import jax
import jax.numpy as jnp
from jax import lax
from jax.experimental import pallas as pl
from jax.experimental.pallas import tpu as pltpu
from jax.experimental.pallas import tpu_sc as plsc

_E = 256
_N = 8192
_R = 8192
_TI = 1024
_CJ = 8192
_NC = 2
_NS = 16
_NW = _NC * _NS
_BPW = _R // _NW
_ICH = 128
_KCH = _BPW // _ICH


def _argmin_kernel(f_ref, e_ref, sf2_ref, se2_ref, idx_ref, et_ref):
    mm = jnp.dot(f_ref[...], e_ref[...], preferred_element_type=jnp.float32)
    d = (sf2_ref[...] + se2_ref[...]) - mm
    m = jnp.min(d, axis=1, keepdims=True)
    cols = lax.broadcasted_iota(jnp.int32, (_TI, _N), 1)
    idx_ref[...] = jnp.min(jnp.where(d == m, cols, _N), axis=1, keepdims=True)
    i = pl.program_id(0)
    et_ref[...] = jnp.swapaxes(e_ref[:, pl.ds(i * _TI, _TI)], 0, 1)


def _argmin_call(flat, embeddings, sf, se, interpret=False):
    return pl.pallas_call(
        _argmin_kernel,
        grid=(_R // _TI,),
        in_specs=[
            pl.BlockSpec((_TI, _E), lambda i: (i, 0)),
            pl.BlockSpec((_E, _N), lambda i: (0, 0)),
            pl.BlockSpec((_TI, 1), lambda i: (i, 0)),
            pl.BlockSpec((1, _N), lambda i: (0, 0)),
        ],
        out_specs=[
            pl.BlockSpec((_TI, 1), lambda i: (i, 0)),
            pl.BlockSpec((_TI, _E), lambda i: (i, 0)),
        ],
        out_shape=[
            jax.ShapeDtypeStruct((_R, 1), jnp.int32),
            jax.ShapeDtypeStruct((_N, _E), jnp.float32),
        ],
        compiler_params=pltpu.CompilerParams(
            dimension_semantics=("arbitrary",),
        ),
        interpret=interpret,
    )(flat, embeddings, sf, se)


def _sc_gather(table, idx3):
    mesh = plsc.VectorSubcoreMesh(core_axis_name="c", subcore_axis_name="s",
                                  num_cores=_NC, num_subcores=_NS)

    @functools.partial(
        pl.kernel,
        out_type=jax.ShapeDtypeStruct((_R, _E), jnp.float32),
        mesh=mesh,
        scratch_types=[
            pltpu.VMEM((_KCH, _ICH), jnp.int32),
            pltpu.VMEM((_BPW, _E), jnp.float32),
            pltpu.SemaphoreType.DMA,
        ],
    )
    def gather_kernel(table_hbm, idx_hbm, out_hbm, idx_v, rows_v, sem):
        wid = lax.axis_index("s") * _NC + lax.axis_index("c")
        pltpu.sync_copy(idx_hbm.at[wid], idx_v)
        copies = [
            pltpu.async_copy(table_hbm.at[idx_v.at[k]],
                             rows_v.at[pl.ds(k * _ICH, _ICH)], sem)
            for k in range(_KCH)
        ]
        for cp in copies:
            cp.wait()
        pltpu.sync_copy(rows_v, out_hbm.at[pl.ds(wid * _BPW, _BPW)])

    return gather_kernel(table, idx3)


def kernel(x, embeddings):
    flat = x.reshape(-1, _E)
    sf2 = jnp.sum(flat ** 2, axis=1, keepdims=True) * 0.5
    se2 = jnp.sum(embeddings ** 2, axis=0, keepdims=True) * 0.5
    idx, et = _argmin_call(flat, embeddings, sf2, se2)
    q = _sc_gather(et, idx.reshape(_NW, _KCH, _ICH))
    return q.reshape(x.shape)


# speedup vs baseline: 1.0836x; 1.0713x over previous
import functools

import jax
import jax.numpy as jnp
from jax import lax
from jax.experimental import pallas as pl
from jax.experimental.pallas import tpu as pltpu
from jax.experimental.pallas import tpu_sc as plsc

_E = 256
_N = 8192
_R = 8192
_TI = 1024
_CJ = 8192
_NC = 2
_NS = 16
_NW = _NC * _NS
_BPW = _R // _NW
_ICH = 128
_KCH = _BPW // _ICH


def _argmin_kernel(f_ref, e_ref, sf2_ref, se2_ref, idx_ref, et_ref):
    mm = jnp.dot(f_ref[...], e_ref[...], preferred_element_type=jnp.float32)
    d = (sf2_ref[...] + se2_ref[...]) - mm
    m = jnp.min(d, axis=1, keepdims=True)
    diff = d - m
    cols = lax.broadcasted_iota(jnp.int32, (1, _N), 1) + (1 << 23)
    cand = lax.bitcast_convert_type(diff, jnp.int32) + cols
    mc = jnp.min(lax.bitcast_convert_type(cand, jnp.float32),
                 axis=1, keepdims=True)
    idx_ref[...] = lax.bitcast_convert_type(mc, jnp.int32) - (1 << 23)
    i = pl.program_id(0)
    et_ref[...] = jnp.swapaxes(e_ref[:, pl.ds(i * _TI, _TI)], 0, 1)


def _argmin_call(flat, embeddings, sf, se, interpret=False):
    return pl.pallas_call(
        _argmin_kernel,
        grid=(_R // _TI,),
        in_specs=[
            pl.BlockSpec((_TI, _E), lambda i: (i, 0)),
            pl.BlockSpec((_E, _N), lambda i: (0, 0)),
            pl.BlockSpec((_TI, 1), lambda i: (i, 0)),
            pl.BlockSpec((1, _N), lambda i: (0, 0)),
        ],
        out_specs=[
            pl.BlockSpec((_TI, 1), lambda i: (i, 0)),
            pl.BlockSpec((_TI, _E), lambda i: (i, 0)),
        ],
        out_shape=[
            jax.ShapeDtypeStruct((_R, 1), jnp.int32),
            jax.ShapeDtypeStruct((_N, _E), jnp.float32),
        ],
        compiler_params=pltpu.CompilerParams(
            dimension_semantics=("arbitrary",),
        ),
        interpret=interpret,
    )(flat, embeddings, sf, se)


def _sc_gather(table, idx3):
    mesh = plsc.VectorSubcoreMesh(core_axis_name="c", subcore_axis_name="s",
                                  num_cores=_NC, num_subcores=_NS)

    @functools.partial(
        pl.kernel,
        out_type=jax.ShapeDtypeStruct((_R, _E), jnp.float32),
        mesh=mesh,
        scratch_types=[
            pltpu.VMEM((_KCH, _ICH), jnp.int32),
            pltpu.VMEM((_BPW, _E), jnp.float32),
            pltpu.SemaphoreType.DMA,
        ],
    )
    def gather_kernel(table_hbm, idx_hbm, out_hbm, idx_v, rows_v, sem):
        wid = lax.axis_index("s") * _NC + lax.axis_index("c")
        pltpu.sync_copy(idx_hbm.at[wid], idx_v)
        copies = [
            pltpu.async_copy(table_hbm.at[idx_v.at[k]],
                             rows_v.at[pl.ds(k * _ICH, _ICH)], sem)
            for k in range(_KCH)
        ]
        for cp in copies:
            cp.wait()
        pltpu.sync_copy(rows_v, out_hbm.at[pl.ds(wid * _BPW, _BPW)])

    return gather_kernel(table, idx3)


def kernel(x, embeddings):
    flat = x.reshape(-1, _E)
    sf2 = jnp.sum(flat ** 2, axis=1, keepdims=True) * 0.5
    se2 = jnp.sum(embeddings ** 2, axis=0, keepdims=True) * 0.5
    idx, et = _argmin_call(flat, embeddings, sf2, se2)
    q = _sc_gather(et, idx.reshape(_NW, _KCH, _ICH))
    return q.reshape(x.shape)
